# baseline (speedup 1.0000x reference)
import functools

import jax
import jax.numpy as jnp
from jax import lax
from jax.experimental import pallas as pl
from jax.experimental.pallas import tpu as pltpu
from jax.experimental.pallas import tpu_sc as plsc

N = 10000
E = 160000
D = 256
H = 128
NPAD = 10240
EP = 163840
NC = 2
NS = 16
EPT = EP // NS
B = 128
NB = EPT // B
K = 4
NCH = NB // K
RPT = NPAD // NS


def _sc_body(xcat_ref, src4_ref, dst3_ref, zrow_ref, zcnt_ref, ones_ref,
             didx_ref, agg_ref, cnt_ref,
             src_ch, dst_ch, rows_a, rows_b, ones_v, didx, sagg, scnt,
             g0, g1, s0, s1, c0, c1):
    c = lax.axis_index("c")
    s = lax.axis_index("s")
    rows = (rows_a, rows_b)
    gsem = (g0, g1)
    ssem = (s0, s1)
    csem = (c0, c1)

    def stage_idx(ch):
        pltpu.sync_copy(src4_ref.at[c, s].at[pl.ds(ch * K, K + 1)], src_ch)
        pltpu.sync_copy(dst3_ref.at[s].at[pl.ds(ch * K, K + 1)], dst_ch)

    def gather_start(j, buf, sem):
        pltpu.async_copy(xcat_ref.at[src_ch.at[j]], buf, sem)

    def gather_wait(buf, sem):
        pltpu.make_async_copy(xcat_ref.at[src_ch.at[0]], buf, sem).wait()

    def scatter_start(j, buf, sem):
        idx = dst_ch.at[j]
        pltpu.async_copy(buf, sagg.at[idx], sem, add=True)

    def count_start(j, sem):
        pltpu.async_copy(ones_v, scnt.at[dst_ch.at[j]], sem, add=True)

    def scatter_wait(buf, sem):
        pltpu.make_async_copy(buf, sagg.at[didx], sem).wait()

    def count_wait(sem):
        pltpu.make_async_copy(ones_v, scnt.at[didx], sem).wait()

    pltpu.sync_copy(ones_ref, ones_v)
    pltpu.sync_copy(didx_ref, didx)
    pltpu.sync_copy(zrow_ref, sagg.at[pl.ds(s * RPT, RPT)])
    pltpu.sync_copy(zcnt_ref, scnt.at[pl.ds(s * RPT, RPT)])
    plsc.subcore_barrier()

    stage_idx(0)
    gather_start(0, rows[0], gsem[0])
    scatter_start_dummy = pltpu.async_copy(rows_b, sagg.at[didx], s1, add=True)
    count_start_dummy = pltpu.async_copy(ones_v, scnt.at[didx], c1, add=True)
    del scatter_start_dummy, count_start_dummy

    def chunk(ch, carry):
        scatter_wait(rows[(K - 1) % 2], ssem[(K - 1) % 2])
        count_wait(csem[(K - 1) % 2])
        gather_wait(rows[0], gsem[0])
        stage_idx(ch)
        for j in range(K):
            buf = rows[j % 2]
            if j >= 1:
                gather_wait(buf, gsem[j % 2])
            scatter_start(j, buf, ssem[j % 2])
            count_start(j, csem[j % 2])
            if j >= 1:
                scatter_wait(rows[(j - 1) % 2], ssem[(j - 1) % 2])
                count_wait(csem[(j - 1) % 2])
            gather_start(j + 1, rows[(j + 1) % 2], gsem[(j + 1) % 2])
        return carry
    lax.fori_loop(0, NCH, chunk, 0)

    scatter_wait(rows[(K - 1) % 2], ssem[(K - 1) % 2])
    count_wait(csem[(K - 1) % 2])
    gather_wait(rows[0], gsem[0])
    plsc.subcore_barrier()

    row = s * RPT
    pltpu.sync_copy(sagg.at[pl.ds(row, RPT)],
                    agg_ref.at[pl.ds(c * NPAD + row, RPT)])
    pltpu.sync_copy(scnt.at[pl.ds(row, RPT)],
                    cnt_ref.at[pl.ds(c * NPAD + row, RPT)])


_sc_aggregate = functools.partial(
    pl.kernel,
    out_type=(
        jax.ShapeDtypeStruct((NC * NPAD, H), jnp.float32),
        jax.ShapeDtypeStruct((NC * NPAD, 16), jnp.float32),
    ),
    mesh=plsc.VectorSubcoreMesh(
        core_axis_name="c", subcore_axis_name="s",
        num_cores=NC, num_subcores=NS),
    compiler_params=pltpu.CompilerParams(use_tc_tiling_on_sc=False),
    scratch_types=[
        pltpu.VMEM((K + 1, B), jnp.int32),
        pltpu.VMEM((K + 1, B), jnp.int32),
        pltpu.VMEM((B, H), jnp.float32),
        pltpu.VMEM((B, H), jnp.float32),
        pltpu.VMEM((B, 16), jnp.float32),
        pltpu.VMEM((B,), jnp.int32),
        pltpu.VMEM_SHARED((NPAD, H), jnp.float32),
        pltpu.VMEM_SHARED((NPAD, 16), jnp.float32),
        pltpu.SemaphoreType.DMA,
        pltpu.SemaphoreType.DMA,
        pltpu.SemaphoreType.DMA,
        pltpu.SemaphoreType.DMA,
        pltpu.SemaphoreType.DMA,
        pltpu.SemaphoreType.DMA,
    ],
)(_sc_body)


R = 80


def _tc_body(lo_ref, hi_ref, cnt_ref, x_ref, wl_ref, wr_ref, b_ref, o_ref):
    cnt = cnt_ref[:, 0:1]
    inv = 1.0 / jnp.maximum(cnt, 1.0)
    agg = jnp.concatenate([lo_ref[...], hi_ref[...]], axis=1) * inv
    acc = jnp.dot(agg, wl_ref[...], preferred_element_type=jnp.float32)
    acc = acc + jnp.dot(x_ref[...], wr_ref[...], preferred_element_type=jnp.float32)
    o_ref[...] = jnp.maximum(acc + b_ref[...], 0.0)


def kernel(x, edge_index, W_l, b_l, W_r):
    src = edge_index[0].astype(jnp.int32)
    dst = edge_index[1].astype(jnp.int32)
    pad = EP - E
    src_p = jnp.concatenate([src, jnp.zeros((pad,), jnp.int32)])
    dst_p = jnp.concatenate([dst, jnp.full((pad,), NPAD - 1, jnp.int32)])
    src3 = jnp.pad(src_p.reshape(NS, NB, B), ((0, 0), (0, 1), (0, 0)))
    dst3 = jnp.pad(dst_p.reshape(NS, NB, B), ((0, 0), (0, 1), (0, 0)),
                   constant_values=NPAD - 1)
    xcat = jnp.concatenate([x[:, :H], x[:, H:]], axis=0)
    src4 = jnp.stack([src3, src3 + N])
    zrow = jnp.zeros((RPT, H), jnp.float32)
    zcnt = jnp.zeros((RPT, 16), jnp.float32)
    ones = jnp.ones((B, 16), jnp.float32)
    didx = jnp.full((B,), NPAD - 1, jnp.int32)

    agg_cat, cnt_cat = _sc_aggregate(xcat, src4, dst3, zrow, zcnt, ones, didx)

    out = pl.pallas_call(
        _tc_body,
        out_shape=jax.ShapeDtypeStruct((N, D), jnp.float32),
        grid=(N // R,),
        in_specs=[
            pl.BlockSpec((R, H), lambda i: (i, 0)),
            pl.BlockSpec((R, H), lambda i: (i + NPAD // R, 0)),
            pl.BlockSpec((R, 16), lambda i: (i, 0)),
            pl.BlockSpec((R, D), lambda i: (i, 0)),
            pl.BlockSpec((D, D), lambda i: (0, 0)),
            pl.BlockSpec((D, D), lambda i: (0, 0)),
            pl.BlockSpec((1, D), lambda i: (0, 0)),
        ],
        out_specs=pl.BlockSpec((R, D), lambda i: (i, 0)),
    )(agg_cat, agg_cat, cnt_cat, x, W_l.T, W_r.T, b_l.reshape(1, D))
    return out


# speedup vs baseline: 1.2160x; 1.2160x over previous
import functools

import jax
import jax.numpy as jnp
from jax import lax
from jax.experimental import pallas as pl
from jax.experimental.pallas import tpu as pltpu
from jax.experimental.pallas import tpu_sc as plsc

N = 10000
E = 160000
D = 256
H = 128
NPAD = 10240
EP = 163840
NC = 2
NS = 16
EPT = EP // NS
B = 128
NB = EPT // B
K = 8
NCH = NB // K
NCH2 = NCH // 2
RPT = NPAD // NS


def _sc_body(xcat_ref, src4_ref, dst3_ref, zrow_ref, zcnt_ref, ones_ref,
             didx_ref, agg_ref, cnt_ref,
             src_ch, dst_ch, rows_a, rows_b, ones_v, didx, sagg, scnt,
             sem_a, sem_b, csem):
    c = lax.axis_index("c")
    s = lax.axis_index("s")

    pltpu.sync_copy(ones_ref, ones_v)
    pltpu.sync_copy(didx_ref, didx)
    pltpu.sync_copy(zrow_ref, sagg.at[pl.ds(s * RPT, RPT)])
    pltpu.sync_copy(zcnt_ref, scnt.at[pl.ds(s * RPT, RPT)])
    plsc.subcore_barrier()

    rows = (rows_a, rows_b)
    sems = (sem_a, sem_b)

    def gather_start(b, buf, sem):
        pltpu.async_copy(xcat_ref.at[src_ch.at[b]], buf, sem)

    def gather_wait(buf, sem):
        pltpu.make_async_copy(xcat_ref.at[src_ch.at[0]], buf, sem).wait()

    def chunk(ch, carry):
        pltpu.sync_copy(src4_ref.at[c, s].at[pl.ds(ch * K, K)], src_ch)
        pltpu.sync_copy(dst3_ref.at[s].at[pl.ds(ch * K, K)], dst_ch)
        gather_start(0, rows[0], sems[0])
        for b in range(K):
            buf, sem = rows[b % 2], sems[b % 2]
            if b + 1 < K:
                gather_start(b + 1, rows[(b + 1) % 2], sems[(b + 1) % 2])
            gather_wait(buf, sem)
            pltpu.sync_copy(buf, sagg.at[dst_ch.at[b]], add=True)
        return carry
    lax.fori_loop(0, NCH, chunk, 0)

    def cchunk(ch, carry):
        pltpu.sync_copy(dst3_ref.at[s].at[pl.ds(ch * K, K)], dst_ch)
        for b in range(K):
            pltpu.async_copy(ones_v, scnt.at[dst_ch.at[b]], csem, add=True)
        for b in range(K):
            pltpu.make_async_copy(ones_v, scnt.at[didx], csem).wait()
        return carry
    lax.fori_loop(c * NCH2, (c + 1) * NCH2, cchunk, 0)
    plsc.subcore_barrier()

    row = s * RPT
    pltpu.sync_copy(sagg.at[pl.ds(row, RPT)],
                    agg_ref.at[pl.ds(c * NPAD + row, RPT)])
    pltpu.sync_copy(scnt.at[pl.ds(row, RPT)],
                    cnt_ref.at[pl.ds(c * NPAD + row, RPT)])


_sc_aggregate = functools.partial(
    pl.kernel,
    out_type=(
        jax.ShapeDtypeStruct((NC * NPAD, H), jnp.float32),
        jax.ShapeDtypeStruct((NC * NPAD, 16), jnp.float32),
    ),
    mesh=plsc.VectorSubcoreMesh(
        core_axis_name="c", subcore_axis_name="s",
        num_cores=NC, num_subcores=NS),
    compiler_params=pltpu.CompilerParams(use_tc_tiling_on_sc=False),
    scratch_types=[
        pltpu.VMEM((K, B), jnp.int32),
        pltpu.VMEM((K, B), jnp.int32),
        pltpu.VMEM((B, H), jnp.float32),
        pltpu.VMEM((B, H), jnp.float32),
        pltpu.VMEM((B, 16), jnp.float32),
        pltpu.VMEM((B,), jnp.int32),
        pltpu.VMEM_SHARED((NPAD, H), jnp.float32),
        pltpu.VMEM_SHARED((NPAD, 16), jnp.float32),
        pltpu.SemaphoreType.DMA,
        pltpu.SemaphoreType.DMA,
        pltpu.SemaphoreType.DMA,
    ],
)(_sc_body)


R = 80


def _tc_body(lo_ref, hi_ref, clo_ref, chi_ref, x_ref, wl_ref, wr_ref, b_ref,
             o_ref):
    cnt = clo_ref[:, 0:1] + chi_ref[:, 0:1]
    inv = 1.0 / jnp.maximum(cnt, 1.0)
    agg = jnp.concatenate([lo_ref[...], hi_ref[...]], axis=1) * inv
    acc = jnp.dot(agg, wl_ref[...], preferred_element_type=jnp.float32)
    acc = acc + jnp.dot(x_ref[...], wr_ref[...], preferred_element_type=jnp.float32)
    o_ref[...] = jnp.maximum(acc + b_ref[...], 0.0)


def kernel(x, edge_index, W_l, b_l, W_r):
    src = edge_index[0].astype(jnp.int32)
    dst = edge_index[1].astype(jnp.int32)
    pad = EP - E
    src_p = jnp.concatenate([src, jnp.zeros((pad,), jnp.int32)])
    dst_p = jnp.concatenate([dst, jnp.full((pad,), NPAD - 1, jnp.int32)])
    src3 = src_p.reshape(NS, NB, B)
    dst3 = dst_p.reshape(NS, NB, B)
    xcat = jnp.concatenate([x[:, :H], x[:, H:]], axis=0)
    src4 = jnp.stack([src3, src3 + N])
    zrow = jnp.zeros((RPT, H), jnp.float32)
    zcnt = jnp.zeros((RPT, 16), jnp.float32)
    ones = jnp.ones((B, 16), jnp.float32)
    didx = jnp.full((B,), NPAD - 1, jnp.int32)

    agg_cat, cnt_cat = _sc_aggregate(xcat, src4, dst3, zrow, zcnt, ones, didx)

    out = pl.pallas_call(
        _tc_body,
        out_shape=jax.ShapeDtypeStruct((N, D), jnp.float32),
        grid=(N // R,),
        in_specs=[
            pl.BlockSpec((R, H), lambda i: (i, 0)),
            pl.BlockSpec((R, H), lambda i: (i + NPAD // R, 0)),
            pl.BlockSpec((R, 16), lambda i: (i, 0)),
            pl.BlockSpec((R, 16), lambda i: (i + NPAD // R, 0)),
            pl.BlockSpec((R, D), lambda i: (i, 0)),
            pl.BlockSpec((D, D), lambda i: (0, 0)),
            pl.BlockSpec((D, D), lambda i: (0, 0)),
            pl.BlockSpec((1, D), lambda i: (0, 0)),
        ],
        out_specs=pl.BlockSpec((R, D), lambda i: (i, 0)),
    )(agg_cat, agg_cat, cnt_cat, cnt_cat, x, W_l.T, W_r.T, b_l.reshape(1, D))
    return out
